# baseline (speedup 1.0000x reference)
import functools

import jax
import jax.numpy as jnp
from jax import lax
from jax.experimental import pallas as pl
from jax.experimental.pallas import tpu as pltpu
from jax.experimental.pallas import tpu_sc as plsc

_LANES = 16


@functools.lru_cache(maxsize=None)
def _build(batch, num_nodes, embed_dim):
    info = plsc.get_sparse_core_info()
    num_cores, num_subcores = info.num_cores, info.num_subcores
    num_workers = num_cores * num_subcores
    b_per_w = batch // num_workers
    chunk = 128
    n_chunks = b_per_w // chunk
    groups_per_chunk = chunk // _LANES

    mesh = plsc.VectorSubcoreMesh(core_axis_name="c", subcore_axis_name="s")

    @functools.partial(
        pl.kernel,
        mesh=mesh,
        compiler_params=pltpu.CompilerParams(
            needs_layout_passes=False, use_tc_tiling_on_sc=False),
        out_type=jax.ShapeDtypeStruct((batch,), jnp.float32),
        scratch_types=[
            pltpu.VMEM((n_chunks, chunk), jnp.int32),
            pltpu.VMEM((n_chunks, chunk), jnp.int32),
            pltpu.VMEM((n_chunks, chunk, embed_dim), jnp.float32),
            pltpu.VMEM((n_chunks, chunk, embed_dim), jnp.float32),
            pltpu.VMEM((b_per_w,), jnp.float32),
        ] + [pltpu.SemaphoreType.DMA for _ in range(n_chunks)],
    )
    def sc_kernel(in_nodes, out_nodes, in_emb, out_emb, scores,
                  idx_a, idx_b, rows_a, rows_b, out_v, *sems):
        wid = lax.axis_index("s") * num_cores + lax.axis_index("c")
        base = wid * b_per_w

        copies = []
        for k in range(n_chunks):
            off = base + k * chunk
            pltpu.sync_copy(in_nodes.at[pl.ds(off, chunk)], idx_a.at[k])
            pltpu.sync_copy(out_nodes.at[pl.ds(off, chunk)], idx_b.at[k])
            ca = pltpu.async_copy(in_emb.at[idx_a.at[k]], rows_a.at[k], sems[k])
            cb = pltpu.async_copy(out_emb.at[idx_b.at[k]], rows_b.at[k], sems[k])
            copies.append((ca, cb))

        iota = lax.broadcasted_iota(jnp.int32, (_LANES,), 0)
        half = embed_dim // 2

        for k in range(n_chunks):
            ca, cb = copies[k]
            ca.wait()
            cb.wait()
            ak = rows_a.at[k]
            bk = rows_b.at[k]

            def group_body(g, _, ak=ak, bk=bk, k=k):
                acc = jnp.zeros((_LANES,), jnp.float32)
                for i in range(_LANES):
                    r = g * _LANES + i
                    a0 = ak[r, pl.ds(0, _LANES)]
                    b0 = bk[r, pl.ds(0, _LANES)]
                    s = a0 * b0
                    for h in range(1, embed_dim // _LANES):
                        a1 = ak[r, pl.ds(h * _LANES, _LANES)]
                        b1 = bk[r, pl.ds(h * _LANES, _LANES)]
                        s = s + a1 * b1
                    tot = jnp.sum(s)
                    acc = jnp.where(iota == i, tot, acc)
                out_v[pl.ds(k * chunk + g * _LANES, _LANES)] = acc
                return 0

            lax.fori_loop(0, groups_per_chunk, group_body, 0)

        pltpu.sync_copy(out_v, scores.at[pl.ds(base, b_per_w)])

    return sc_kernel


def kernel(input_nodes, output_nodes, in_embeddings, out_embeddings):
    batch = input_nodes.shape[0]
    num_nodes, embed_dim = in_embeddings.shape
    fn = _build(batch, num_nodes, embed_dim)
    return fn(input_nodes.astype(jnp.int32), output_nodes.astype(jnp.int32),
              in_embeddings, out_embeddings)


# speedup vs baseline: 3.7228x; 3.7228x over previous
import functools

import jax
import jax.numpy as jnp
from jax import lax
from jax.experimental import pallas as pl
from jax.experimental.pallas import tpu as pltpu
from jax.experimental.pallas import tpu_sc as plsc

_LANES = 16
_WIN = 128
_RING = 8


@functools.lru_cache(maxsize=None)
def _build(batch, num_nodes, embed_dim):
    info = plsc.get_sparse_core_info()
    num_cores, num_subcores = info.num_cores, info.num_subcores
    num_workers = num_cores * num_subcores
    b_per_w = batch // num_workers
    groups = b_per_w // _LANES
    n_main = num_nodes // _WIN
    tail = num_nodes % _WIN
    tail_start = n_main * _WIN

    mesh = plsc.VectorSubcoreMesh(core_axis_name="c", subcore_axis_name="s")

    @functools.partial(
        pl.kernel,
        mesh=mesh,
        compiler_params=pltpu.CompilerParams(needs_layout_passes=False),
        out_type=jax.ShapeDtypeStruct((batch,), jnp.float32),
        scratch_types=[
            pltpu.VMEM((b_per_w,), jnp.int32),
            pltpu.VMEM((b_per_w,), jnp.int32),
            pltpu.VMEM((_RING, embed_dim, _WIN), jnp.float32),
            pltpu.VMEM((_RING, embed_dim, _WIN), jnp.float32),
            pltpu.VMEM((2, embed_dim, max(tail, 1)), jnp.float32),
            pltpu.VMEM((b_per_w,), jnp.float32),
            pltpu.SemaphoreType.DMA,
        ] + [pltpu.SemaphoreType.DMA for _ in range(_RING)],
    )
    def sc_kernel(in_nodes, out_nodes, in_emb_t, out_emb_t, scores,
                  idx_a, idx_b, ring_a, ring_b, tails, out_v, sem_t, *sems):
        wid = lax.axis_index("s") * num_cores + lax.axis_index("c")
        base = wid * b_per_w

        pltpu.sync_copy(in_nodes.at[pl.ds(base, b_per_w)], idx_a)
        pltpu.sync_copy(out_nodes.at[pl.ds(base, b_per_w)], idx_b)

        if tail:
            ta = pltpu.async_copy(
                in_emb_t.at[:, pl.ds(tail_start, tail)], tails.at[0], sem_t)
            tb = pltpu.async_copy(
                out_emb_t.at[:, pl.ds(tail_start, tail)], tails.at[1], sem_t)
            ta.wait()
            tb.wait()

        iota = lax.broadcasted_iota(jnp.int32, (_LANES,), 0)
        half = embed_dim // _LANES

        def issue(va, vb, e, slot):
            wa = jnp.minimum(va[e] // _WIN, n_main - 1)
            wb = jnp.minimum(vb[e] // _WIN, n_main - 1)
            ca = pltpu.async_copy(
                in_emb_t.at[:, pl.ds(wa * _WIN, _WIN)], ring_a.at[slot],
                sems[slot])
            cb = pltpu.async_copy(
                out_emb_t.at[:, pl.ds(wb * _WIN, _WIN)], ring_b.at[slot],
                sems[slot])
            return ca, cb

        def lookup(ring, tail_blk, v, e):
            i = v[e]
            col = jnp.full((_LANES,), i & (_WIN - 1), jnp.int32)
            rows = []
            for h in range(half):
                rows.append(plsc.load_gather(ring, [iota + h * _LANES, col]))
            if tail:
                tcol_s = jnp.minimum(jnp.maximum(i - tail_start, 0), tail - 1)
                tcol = jnp.full((_LANES,), tcol_s, jnp.int32)
                is_tail = i >= tail_start
                for h in range(half):
                    t = plsc.load_gather(tail_blk, [iota + h * _LANES, tcol])
                    rows[h] = jnp.where(is_tail, t, rows[h])
            return rows

        def group_body(g, _):
            gbase = g * _LANES
            va = idx_a[pl.ds(gbase, _LANES)]
            vb = idx_b[pl.ds(gbase, _LANES)]
            copies = [issue(va, vb, e, e) for e in range(_RING)]
            acc = jnp.zeros((_LANES,), jnp.float32)
            for e in range(_LANES):
                slot = e % _RING
                ca, cb = copies[slot]
                ca.wait()
                cb.wait()
                ar = lookup(ring_a.at[slot], tails.at[0], va, e)
                br = lookup(ring_b.at[slot], tails.at[1], vb, e)
                prod = ar[0] * br[0]
                for h in range(1, half):
                    prod = prod + ar[h] * br[h]
                if e + _RING < _LANES:
                    copies[slot] = issue(va, vb, e + _RING, slot)
                acc = jnp.where(iota == e, jnp.sum(prod), acc)
            out_v[pl.ds(gbase, _LANES)] = acc
            return 0

        lax.fori_loop(0, groups, group_body, 0)

        pltpu.sync_copy(out_v, scores.at[pl.ds(base, b_per_w)])

    return sc_kernel


def kernel(input_nodes, output_nodes, in_embeddings, out_embeddings):
    batch = input_nodes.shape[0]
    num_nodes, embed_dim = in_embeddings.shape
    fn = _build(batch, num_nodes, embed_dim)
    return fn(input_nodes.astype(jnp.int32), output_nodes.astype(jnp.int32),
              jnp.swapaxes(in_embeddings, 0, 1),
              jnp.swapaxes(out_embeddings, 0, 1))
